# Initial kernel scaffold; baseline (speedup 1.0000x reference)
import jax
import jax.numpy as jnp
from jax.experimental import pallas as pl


def kernel(indices, table):
    raise NotImplementedError("write your pallas kernel here")


# speedup vs baseline: 3.5598x; 3.5598x over previous
import functools

import jax
import jax.numpy as jnp
from jax import lax
from jax.experimental import pallas as pl
from jax.experimental.pallas import tpu as pltpu
from jax.experimental.pallas import tpu_sc as plsc

N_NODES = 10000
DIM = 128

NC = 2
NS = 16
NW = NC * NS

B = 4096
L = 50
N = B * L
PER_W = N // NW
G = 128
GROUP = 5
STEPS = PER_W // (G * GROUP)


@functools.cache
def _make_sc_gather():
    mesh = plsc.VectorSubcoreMesh(
        core_axis_name="c", subcore_axis_name="s", num_cores=NC, num_subcores=NS
    )

    @functools.partial(
        pl.kernel,
        out_type=jax.ShapeDtypeStruct((N, DIM), jnp.float32),
        mesh=mesh,
        scratch_types=[
            pltpu.VMEM((PER_W // G, G), jnp.int32),
            pltpu.VMEM((GROUP * G, DIM), jnp.float32),
            pltpu.SemaphoreType.DMA,
        ],
    )
    def k(idx_hbm, table_hbm, out_hbm, idx_v, rows_v, sem):
        wid = lax.axis_index("s") * NC + lax.axis_index("c")
        pltpu.sync_copy(idx_hbm.at[wid], idx_v)
        base = wid * PER_W

        @pl.loop(0, STEPS)
        def _step(step):
            copies = []
            for j in range(GROUP):
                cp = pltpu.async_copy(
                    table_hbm.at[idx_v.at[step * GROUP + j]],
                    rows_v.at[pl.ds(j * G, G)],
                    sem,
                )
                copies.append(cp)
            for cp in copies:
                cp.wait()
            pltpu.sync_copy(
                rows_v, out_hbm.at[pl.ds(base + step * (GROUP * G), GROUP * G)]
            )

    return k


@jax.jit
def kernel(indices, table):
    idx3d = indices.astype(jnp.int32).reshape(NW, PER_W // G, G)
    out = _make_sc_gather()(idx3d, table)
    return out.reshape(B, L, DIM)


# speedup vs baseline: 3.6007x; 1.0115x over previous
import functools

import jax
import jax.numpy as jnp
from jax import lax
from jax.experimental import pallas as pl
from jax.experimental.pallas import tpu as pltpu
from jax.experimental.pallas import tpu_sc as plsc

N_NODES = 10000
DIM = 128

NC = 2
NS = 16
NW = NC * NS

B = 4096
L = 50
N = B * L
PER_W = N // NW
G = 128
GROUP = 5
STEPS = PER_W // (G * GROUP)


@functools.cache
def _make_sc_gather():
    mesh = plsc.VectorSubcoreMesh(
        core_axis_name="c", subcore_axis_name="s", num_cores=NC, num_subcores=NS
    )

    @functools.partial(
        pl.kernel,
        out_type=jax.ShapeDtypeStruct((N, DIM), jnp.float32),
        mesh=mesh,
        scratch_types=[
            pltpu.VMEM((PER_W // G, G), jnp.int32),
            pltpu.VMEM((GROUP * G, DIM), jnp.float32),
            [pltpu.SemaphoreType.DMA] * GROUP,
            [pltpu.SemaphoreType.DMA] * GROUP,
        ],
    )
    def k(idx_hbm, table_hbm, out_hbm, idx_v, rows_v, gsems, wsems):
        wid = lax.axis_index("s") * NC + lax.axis_index("c")
        pltpu.sync_copy(idx_hbm.at[wid], idx_v)
        base = wid * PER_W

        @pl.loop(0, STEPS)
        def _step(step):
            for b in range(GROUP):
                @pl.when(step > 0)
                def _wait_write(b=b):
                    pltpu.make_async_copy(
                        rows_v.at[pl.ds(b * G, G)],
                        out_hbm.at[pl.ds(base, G)],
                        wsems[b],
                    ).wait()

                pltpu.async_copy(
                    table_hbm.at[idx_v.at[step * GROUP + b]],
                    rows_v.at[pl.ds(b * G, G)],
                    gsems[b],
                )
            for b in range(GROUP):
                pltpu.make_async_copy(
                    table_hbm.at[idx_v.at[step * GROUP + b]],
                    rows_v.at[pl.ds(b * G, G)],
                    gsems[b],
                ).wait()
                pltpu.async_copy(
                    rows_v.at[pl.ds(b * G, G)],
                    out_hbm.at[pl.ds(base + (step * GROUP + b) * G, G)],
                    wsems[b],
                )

        for b in range(GROUP):
            pltpu.make_async_copy(
                rows_v.at[pl.ds(b * G, G)],
                out_hbm.at[pl.ds(base, G)],
                wsems[b],
            ).wait()

    return k


@jax.jit
def kernel(indices, table):
    idx3d = indices.astype(jnp.int32).reshape(NW, PER_W // G, G)
    out = _make_sc_gather()(idx3d, table)
    return out.reshape(B, L, DIM)


# speedup vs baseline: 7.7235x; 2.1450x over previous
import functools

import jax
import jax.numpy as jnp
from jax import lax
from jax.experimental import pallas as pl
from jax.experimental.pallas import tpu as pltpu
from jax.experimental.pallas import tpu_sc as plsc

N_NODES = 10000
DIM = 128

NC = 2
NS = 16
NW = NC * NS

B = 4096
L = 50
LP = 56
PB = B // NW
NBUF = 4
STEPS = PB // NBUF


@functools.cache
def _make_sc_gather():
    mesh = plsc.VectorSubcoreMesh(
        core_axis_name="c", subcore_axis_name="s", num_cores=NC, num_subcores=NS
    )

    @functools.partial(
        pl.kernel,
        out_type=jax.ShapeDtypeStruct((B, L, DIM), jnp.float32),
        mesh=mesh,
        scratch_types=[
            pltpu.VMEM((PB * LP,), jnp.int32),
            [pltpu.VMEM((LP, DIM), jnp.float32)] * NBUF,
            pltpu.VMEM_SHARED((N_NODES, DIM), jnp.float32),
            [pltpu.SemaphoreType.DMA] * NBUF,
            [pltpu.SemaphoreType.DMA] * NBUF,
        ],
        compiler_params=pltpu.CompilerParams(use_tc_tiling_on_sc=True),
    )
    def k(idx_hbm, table_hbm, out_hbm, idx_v, bufs, table_sh, gsems, wsems):
        wid = lax.axis_index("s") * NC + lax.axis_index("c")

        @pl.when(lax.axis_index("s") == 0)
        def _stage():
            pltpu.sync_copy(table_hbm, table_sh)

        pltpu.sync_copy(idx_hbm.at[pl.ds(wid * PB * LP, PB * LP)], idx_v)
        plsc.subcore_barrier()
        base = wid * PB

        @pl.loop(0, STEPS)
        def _step(step):
            for sl in range(NBUF):
                @pl.when(step > 0)
                def _wait_write(sl=sl):
                    pltpu.make_async_copy(
                        bufs[sl].at[pl.ds(0, L)],
                        out_hbm.at[base],
                        wsems[sl],
                    ).wait()

                pltpu.async_copy(
                    table_sh.at[idx_v.at[pl.ds((step * NBUF + sl) * LP, L)]],
                    bufs[sl].at[pl.ds(0, L)],
                    gsems[sl],
                )
            for sl in range(NBUF):
                jb = step * NBUF + sl
                pltpu.make_async_copy(
                    table_sh.at[idx_v.at[pl.ds(jb * LP, L)]],
                    bufs[sl].at[pl.ds(0, L)],
                    gsems[sl],
                ).wait()
                pltpu.async_copy(
                    bufs[sl].at[pl.ds(0, L)],
                    out_hbm.at[base + jb],
                    wsems[sl],
                )

        for sl in range(NBUF):
            pltpu.make_async_copy(
                bufs[sl].at[pl.ds(0, L)],
                out_hbm.at[base],
                wsems[sl],
            ).wait()

    return k


@jax.jit
def kernel(indices, table):
    idx = indices.astype(jnp.int32)
    idx_pad = jnp.pad(idx, ((0, 0), (0, LP - L))).reshape(-1)
    return _make_sc_gather()(idx_pad, table)


# speedup vs baseline: 11.9472x; 1.5469x over previous
import functools

import jax
import jax.numpy as jnp
from jax import lax
from jax.experimental import pallas as pl
from jax.experimental.pallas import tpu as pltpu
from jax.experimental.pallas import tpu_sc as plsc

N_NODES = 10000
DIM = 128

NC = 2
NS = 16
NW = NC * NS

B = 4096
L = 50
N = B * L
PER_W = N // NW
G = 128
NBUF = 2
STEPS = PER_W // (G * NBUF)


@functools.cache
def _make_sc_gather():
    mesh = plsc.VectorSubcoreMesh(
        core_axis_name="c", subcore_axis_name="s", num_cores=NC, num_subcores=NS
    )

    @functools.partial(
        pl.kernel,
        out_type=jax.ShapeDtypeStruct((N, DIM), jnp.float32),
        mesh=mesh,
        scratch_types=[
            pltpu.VMEM((PER_W,), jnp.int32),
            [pltpu.VMEM((G, DIM), jnp.float32)] * NBUF,
            pltpu.VMEM_SHARED((N_NODES, DIM), jnp.float32),
            [pltpu.SemaphoreType.DMA] * NBUF,
            [pltpu.SemaphoreType.DMA] * NBUF,
        ],
        compiler_params=pltpu.CompilerParams(use_tc_tiling_on_sc=True),
    )
    def k(idx_hbm, table_hbm, out2d, idx_v, bufs, table_sh, gsems, wsems):
        wid = lax.axis_index("s") * NC + lax.axis_index("c")

        @pl.when(lax.axis_index("s") == 0)
        def _stage():
            pltpu.sync_copy(table_hbm, table_sh)

        pltpu.sync_copy(idx_hbm.at[pl.ds(wid * PER_W, PER_W)], idx_v)
        plsc.subcore_barrier()
        base = wid * PER_W

        @pl.loop(0, STEPS)
        def _step(step):
            for sl in range(NBUF):
                @pl.when(step > 0)
                def _wait_write(sl=sl):
                    pltpu.make_async_copy(
                        bufs[sl], out2d.at[pl.ds(base, G)], wsems[sl]
                    ).wait()

                pltpu.async_copy(
                    table_sh.at[idx_v.at[pl.ds((step * NBUF + sl) * G, G)]],
                    bufs[sl],
                    gsems[sl],
                )
            for sl in range(NBUF):
                g = step * NBUF + sl
                pltpu.make_async_copy(
                    table_sh.at[idx_v.at[pl.ds(g * G, G)]], bufs[sl], gsems[sl]
                ).wait()
                pltpu.async_copy(
                    bufs[sl], out2d.at[pl.ds(base + g * G, G)], wsems[sl]
                )

        for sl in range(NBUF):
            pltpu.make_async_copy(
                bufs[sl], out2d.at[pl.ds(base, G)], wsems[sl]
            ).wait()

    return k


@jax.jit
def kernel(indices, table):
    idx_t = indices.astype(jnp.int32).T.reshape(-1)
    out_t = _make_sc_gather()(idx_t, table)
    return out_t.reshape(L, B, DIM).transpose(1, 0, 2)


# speedup vs baseline: 15.8924x; 1.3302x over previous
import functools

import jax
import jax.numpy as jnp
from jax import lax
from jax.experimental import pallas as pl
from jax.experimental.pallas import tpu as pltpu
from jax.experimental.pallas import tpu_sc as plsc

N_NODES = 10000
DIM = 128

NC = 2
NS = 16
NW = NC * NS

B = 4096
L = 50
N = B * L
PER_W = N // NW
G = 80
NBUF = 4
STEPS = PER_W // (G * NBUF)


@functools.cache
def _make_sc_gather():
    mesh = plsc.VectorSubcoreMesh(
        core_axis_name="c", subcore_axis_name="s", num_cores=NC, num_subcores=NS
    )

    @functools.partial(
        pl.kernel,
        out_type=jax.ShapeDtypeStruct((N, DIM), jnp.float32),
        mesh=mesh,
        scratch_types=[
            pltpu.VMEM((PER_W,), jnp.int32),
            [pltpu.VMEM((G, DIM), jnp.float32)] * NBUF,
            pltpu.VMEM_SHARED((N_NODES, DIM), jnp.float32),
            [pltpu.SemaphoreType.DMA] * NBUF,
            [pltpu.SemaphoreType.DMA] * NBUF,
        ],
        compiler_params=pltpu.CompilerParams(use_tc_tiling_on_sc=True),
    )
    def k(idx_hbm, table_hbm, out2d, idx_v, bufs, table_sh, gsems, wsems):
        wid = lax.axis_index("s") * NC + lax.axis_index("c")

        @pl.when(lax.axis_index("s") == 0)
        def _stage():
            pltpu.sync_copy(table_hbm, table_sh)

        pltpu.sync_copy(idx_hbm.at[pl.ds(wid * PER_W, PER_W)], idx_v)
        plsc.subcore_barrier()
        base = wid * PER_W

        @pl.loop(0, STEPS)
        def _step(step):
            for sl in range(NBUF):
                @pl.when(step > 0)
                def _wait_write(sl=sl):
                    pltpu.make_async_copy(
                        bufs[sl], out2d.at[pl.ds(base, G)], wsems[sl]
                    ).wait()

                pltpu.async_copy(
                    table_sh.at[idx_v.at[pl.ds((step * NBUF + sl) * G, G)]],
                    bufs[sl],
                    gsems[sl],
                )
            for sl in range(NBUF):
                g = step * NBUF + sl
                pltpu.make_async_copy(
                    table_sh.at[idx_v.at[pl.ds(g * G, G)]], bufs[sl], gsems[sl]
                ).wait()
                pltpu.async_copy(
                    bufs[sl], out2d.at[pl.ds(base + g * G, G)], wsems[sl]
                )

        for sl in range(NBUF):
            pltpu.make_async_copy(
                bufs[sl], out2d.at[pl.ds(base, G)], wsems[sl]
            ).wait()

    return k


@jax.jit
def kernel(indices, table):
    idx_t = indices.astype(jnp.int32).T.reshape(-1)
    out_t = _make_sc_gather()(idx_t, table)
    return out_t.reshape(L, B, DIM).transpose(1, 0, 2)
